# Initial kernel scaffold; baseline (speedup 1.0000x reference)
import jax
import jax.numpy as jnp
from jax.experimental import pallas as pl


def kernel(token_ids, table):
    raise NotImplementedError("write your pallas kernel here")


# speedup vs baseline: 1.1028x; 1.1028x over previous
import functools
import jax
import jax.numpy as jnp
from jax import lax
from jax.experimental import pallas as pl
from jax.experimental.pallas import tpu as pltpu
from jax.experimental.pallas import tpu_sc as plsc

NC = 2
NS = 16
NW = NC * NS
CW = 128
NBUF = 8


@jax.jit
def kernel(token_ids, table):
    B, L = token_ids.shape
    V, D = table.shape
    btot = B * L
    assert btot % (NW * CW) == 0
    chunks_per_w = btot // (NW * CW)
    rows_per_w = chunks_per_w * CW
    idx2d = token_ids.reshape(NW * chunks_per_w, CW).astype(jnp.int32)

    mesh = plsc.VectorSubcoreMesh(
        core_axis_name="c", subcore_axis_name="s",
        num_cores=NC, num_subcores=NS)

    @functools.partial(
        pl.kernel,
        mesh=mesh,
        out_type=jax.ShapeDtypeStruct((btot, D), jnp.float32),
        scratch_types=[
            pltpu.VMEM((chunks_per_w, CW), jnp.int32),
            pltpu.VMEM((NBUF * CW, D), jnp.float32),
            pltpu.SemaphoreType.DMA,
        ],
        compiler_params=pltpu.CompilerParams(use_tc_tiling_on_sc=False),
    )
    def emb(table_hbm, idx_hbm, out_hbm, idx_v, rows_v, sem):
        wid = lax.axis_index("s") * NC + lax.axis_index("c")
        pltpu.sync_copy(idx_hbm.at[pl.ds(wid * chunks_per_w, chunks_per_w)],
                        idx_v)
        out_base = wid * rows_per_w

        @pl.loop(0, chunks_per_w, step=NBUF)
        def _(t):
            copies = []
            for b in range(NBUF):
                copies.append(pltpu.async_copy(
                    table_hbm.at[idx_v.at[t + b]],
                    rows_v.at[pl.ds(b * CW, CW)],
                    sem))
            for c in copies:
                c.wait()
            pltpu.sync_copy(rows_v,
                            out_hbm.at[pl.ds(out_base + t * CW, NBUF * CW)])

    out = emb(table, idx2d)
    return out.reshape(B, L, D)


# speedup vs baseline: 1.1104x; 1.0069x over previous
import functools
import jax
import jax.numpy as jnp
from jax import lax
from jax.experimental import pallas as pl
from jax.experimental.pallas import tpu as pltpu
from jax.experimental.pallas import tpu_sc as plsc

NC = 2
NS = 16
NW = NC * NS
CW = 128
NBUF = 10


@jax.jit
def kernel(token_ids, table):
    B, L = token_ids.shape
    V, D = table.shape
    btot = B * L
    assert btot % (NW * CW * NBUF * 2) == 0
    chunks_per_w = btot // (NW * CW)
    rows_per_w = chunks_per_w * CW
    rpr = NBUF * CW
    T = chunks_per_w // NBUF
    idx2d = token_ids.reshape(NW * chunks_per_w, CW).astype(jnp.int32)

    mesh = plsc.VectorSubcoreMesh(
        core_axis_name="c", subcore_axis_name="s",
        num_cores=NC, num_subcores=NS)

    @functools.partial(
        pl.kernel,
        mesh=mesh,
        out_type=jax.ShapeDtypeStruct((btot, D), jnp.float32),
        scratch_types=[
            pltpu.VMEM((chunks_per_w, CW), jnp.int32),
            pltpu.VMEM((rpr, D), jnp.float32),
            pltpu.VMEM((rpr, D), jnp.float32),
            pltpu.SemaphoreType.DMA,
            pltpu.SemaphoreType.DMA,
            pltpu.SemaphoreType.DMA,
            pltpu.SemaphoreType.DMA,
        ],
        compiler_params=pltpu.CompilerParams(use_tc_tiling_on_sc=False),
    )
    def emb(table_hbm, idx_hbm, out_hbm, idx_v, rows_a, rows_b,
            gsem_a, gsem_b, osem_a, osem_b):
        wid = lax.axis_index("s") * NC + lax.axis_index("c")
        pltpu.sync_copy(idx_hbm.at[pl.ds(wid * chunks_per_w, chunks_per_w)],
                        idx_v)
        out_base = wid * rows_per_w

        def fire_gathers(rnd, rows, sem):
            for b in range(NBUF):
                pltpu.async_copy(table_hbm.at[idx_v.at[rnd * NBUF + b]],
                                 rows.at[pl.ds(b * CW, CW)], sem)

        def wait_gathers(rnd, rows, sem):
            for b in range(NBUF):
                pltpu.make_async_copy(table_hbm.at[idx_v.at[rnd * NBUF + b]],
                                      rows.at[pl.ds(b * CW, CW)], sem).wait()

        def out_slice(rnd):
            return out_hbm.at[pl.ds(out_base + rnd * rpr, rpr)]

        fire_gathers(0, rows_a, gsem_a)

        @pl.loop(0, T, step=2)
        def _(t):
            wait_gathers(t, rows_a, gsem_a)

            @pl.when(t > 0)
            def _():
                pltpu.make_async_copy(rows_b, out_slice(t - 1), osem_b).wait()

            fire_gathers(t + 1, rows_b, gsem_b)
            pltpu.async_copy(rows_a, out_slice(t), osem_a)

            wait_gathers(t + 1, rows_b, gsem_b)

            @pl.when(t + 2 < T)
            def _():
                pltpu.make_async_copy(rows_a, out_slice(t), osem_a).wait()
                fire_gathers(t + 2, rows_a, gsem_a)

            pltpu.async_copy(rows_b, out_slice(t + 1), osem_b)

        pltpu.make_async_copy(rows_a, out_slice(T - 2), osem_a).wait()
        pltpu.make_async_copy(rows_b, out_slice(T - 1), osem_b).wait()

    out = emb(table, idx2d)
    return out.reshape(B, L, D)


# speedup vs baseline: 1.4809x; 1.3337x over previous
import functools
import jax
import jax.numpy as jnp
from jax import lax
from jax.experimental import pallas as pl
from jax.experimental.pallas import tpu as pltpu
from jax.experimental.pallas import tpu_sc as plsc

NC = 2
NS = 16
NW = NC * NS
CB = 128


@jax.jit
def kernel(token_ids, table):
    B, L = token_ids.shape
    V, D = table.shape
    assert D == 32 and B % (NW * CB) == 0
    DT, DR = D // 8, 8
    cb_per_w = B // (NW * CB)
    toks_per_w = cb_per_w * CB * L
    chunks = cb_per_w * L
    idx_flat = token_ids.reshape(B * L // CB, CB).astype(jnp.int32)

    mesh = plsc.VectorSubcoreMesh(
        core_axis_name="c", subcore_axis_name="s",
        num_cores=NC, num_subcores=NS)

    @functools.partial(
        pl.kernel,
        mesh=mesh,
        out_type=jax.ShapeDtypeStruct((L, DT, B // CB, DR, CB), jnp.float32),
        scratch_types=[
            pltpu.VMEM((toks_per_w // CB, CB), jnp.int32),
            pltpu.VMEM((CB,), jnp.int32),
            pltpu.VMEM((CB,), jnp.int32),
            pltpu.VMEM((CB, D), jnp.float32),
            pltpu.VMEM((CB, D), jnp.float32),
            pltpu.VMEM((DT, DR, CB), jnp.float32),
            pltpu.SemaphoreType.DMA,
            pltpu.SemaphoreType.DMA,
        ],
        compiler_params=pltpu.CompilerParams(
            use_tc_tiling_on_sc=False, needs_layout_passes=False),
    )
    def emb(table_hbm, idx_hbm, out_hbm, idx_v, ic_a, ic_b, g_a, g_b,
            tbuf, gsem_a, gsem_b):
        wid = lax.axis_index("s") * NC + lax.axis_index("c")
        pltpu.sync_copy(
            idx_hbm.at[pl.ds(wid * (toks_per_w // CB), toks_per_w // CB)],
            idx_v)
        lanes = lax.iota(jnp.int32, 16)
        svecs = [lanes + (k * 16) for k in range(8)]
        pvecs = [s * L for s in svecs]

        def build_idx(m, ic):
            base = (m // L) * (CB * L) + (m % L)
            for k in range(8):
                p = pvecs[k] + base
                ic[pl.ds(k * 16, 16)] = plsc.load_gather(
                    idx_v, [p >> 7, p & (CB - 1)])

        def fire(ic, g, sem):
            pltpu.async_copy(table_hbm.at[ic], g, sem)

        def wait(ic, g, sem):
            pltpu.make_async_copy(table_hbm.at[ic], g, sem).wait()

        def emit(m, g):
            for dt in range(DT):
                for dr in range(DR):
                    dvec = jnp.full((16,), dt * 8 + dr, jnp.int32)
                    for k in range(8):
                        tbuf[dt, dr, pl.ds(k * 16, 16)] = plsc.load_gather(
                            g, [svecs[k], dvec])
            c = wid * cb_per_w + m // L
            pltpu.sync_copy(tbuf, out_hbm.at[m % L, :, c])

        build_idx(0, ic_a)
        fire(ic_a, g_a, gsem_a)

        @pl.loop(0, chunks, step=2)
        def _(m):
            build_idx(m + 1, ic_b)
            fire(ic_b, g_b, gsem_b)
            wait(ic_a, g_a, gsem_a)
            emit(m, g_a)

            @pl.when(m + 2 < chunks)
            def _():
                build_idx(m + 2, ic_a)
                fire(ic_a, g_a, gsem_a)

            wait(ic_b, g_b, gsem_b)
            emit(m + 1, g_b)

    out5 = emb(table, idx_flat)
    return out5.transpose(2, 4, 0, 1, 3).reshape(B, L, D)


# speedup vs baseline: 2.0929x; 1.4132x over previous
import functools
import jax
import jax.numpy as jnp
from jax import lax
from jax.experimental import pallas as pl
from jax.experimental.pallas import tpu as pltpu
from jax.experimental.pallas import tpu_sc as plsc

NC = 2
NS = 16
NW = NC * NS
CB = 128


@jax.jit
def kernel(token_ids, table):
    B, L = token_ids.shape
    V, D = table.shape
    assert D == 32 and B % (NW * CB) == 0
    DT, DR = D // 8, 8
    cb_per_w = B // (NW * CB)
    toks_per_w = cb_per_w * CB * L
    chunks = cb_per_w * L
    idx_flat = token_ids.reshape(B * L // CB, CB).astype(jnp.int32)

    mesh = plsc.VectorSubcoreMesh(
        core_axis_name="c", subcore_axis_name="s",
        num_cores=NC, num_subcores=NS)

    @functools.partial(
        pl.kernel,
        mesh=mesh,
        out_type=jax.ShapeDtypeStruct((L, DT, B // CB, DR, CB), jnp.float32),
        scratch_types=[
            pltpu.VMEM((toks_per_w // CB, CB), jnp.int32),
            pltpu.VMEM((CB,), jnp.int32),
            pltpu.VMEM((CB,), jnp.int32),
            pltpu.VMEM((CB, D), jnp.float32),
            pltpu.VMEM((CB, D), jnp.float32),
            pltpu.VMEM((DT, DR, CB), jnp.float32),
            pltpu.SemaphoreType.DMA,
            pltpu.SemaphoreType.DMA,
        ],
        compiler_params=pltpu.CompilerParams(
            use_tc_tiling_on_sc=False, needs_layout_passes=False),
    )
    def emb(table_hbm, idx_hbm, out_hbm, idx_v, ic_a, ic_b, g_a, g_b,
            tbuf, gsem_a, gsem_b):
        wid = lax.axis_index("s") * NC + lax.axis_index("c")
        pltpu.sync_copy(
            idx_hbm.at[pl.ds(wid * (toks_per_w // CB), toks_per_w // CB)],
            idx_v)
        lanes = lax.iota(jnp.int32, 16)
        svecs = [lanes + (k * 16) for k in range(8)]
        pvecs = [s * L for s in svecs]

        def build_idx(m, ic):
            base = (m // L) * (CB * L) + (m % L)
            for k in range(8):
                p = pvecs[k] + base
                ic[pl.ds(k * 16, 16)] = plsc.load_gather(
                    idx_v, [p >> 7, p & (CB - 1)])

        def fire(ic, g, sem):
            pltpu.async_copy(table_hbm.at[ic], g, sem)

        def wait(ic, g, sem):
            pltpu.make_async_copy(table_hbm.at[ic], g, sem).wait()

        def emit(m, g):
            @plsc.parallel_loop(0, D, unroll=4)
            def _(d):
                dvec = jnp.full((16,), d, jnp.int32)
                for k in range(8):
                    tbuf[d >> 3, d & 7, pl.ds(k * 16, 16)] = plsc.load_gather(
                        g, [svecs[k], dvec])
            c = wid * cb_per_w + m // L
            pltpu.sync_copy(tbuf, out_hbm.at[m % L, :, c])

        build_idx(0, ic_a)
        fire(ic_a, g_a, gsem_a)

        @pl.loop(0, chunks, step=2)
        def _(m):
            build_idx(m + 1, ic_b)
            fire(ic_b, g_b, gsem_b)
            wait(ic_a, g_a, gsem_a)
            emit(m, g_a)

            @pl.when(m + 2 < chunks)
            def _():
                build_idx(m + 2, ic_a)
                fire(ic_a, g_a, gsem_a)

            wait(ic_b, g_b, gsem_b)
            emit(m + 1, g_b)

    out5 = emb(table, idx_flat)
    return out5.transpose(2, 4, 0, 1, 3).reshape(B, L, D)
